# Initial kernel scaffold; baseline (speedup 1.0000x reference)
import jax
import jax.numpy as jnp
from jax.experimental import pallas as pl


def kernel(sen_idx, table):
    raise NotImplementedError("write your pallas kernel here")


# speedup vs baseline: 2.9584x; 2.9584x over previous
import functools

import jax
import jax.numpy as jnp
from jax import lax
from jax.experimental import pallas as pl
from jax.experimental.pallas import tpu as pltpu
from jax.experimental.pallas import tpu_sc as plsc

EMBED_DIM = 128
SEQ = 50
BATCH = 4096
B_TOTAL = BATCH * SEQ
IDX_COLS = 128
IDX_ROWS = B_TOTAL // IDX_COLS
NUM_WORKERS = 32
ROWS_PER_W = IDX_ROWS // NUM_WORKERS
CHUNK = 128


def _sc_embed(idx2d, table):
    mesh = plsc.VectorSubcoreMesh(core_axis_name="c", subcore_axis_name="s")

    @functools.partial(
        pl.kernel,
        mesh=mesh,
        out_type=[
            jax.ShapeDtypeStruct((B_TOTAL, EMBED_DIM), jnp.float32),
            jax.ShapeDtypeStruct((NUM_WORKERS, ROWS_PER_W, IDX_COLS), jnp.int32),
        ],
        scratch_types=[
            pltpu.VMEM((ROWS_PER_W, IDX_COLS), jnp.int32),
            pltpu.VMEM((ROWS_PER_W, IDX_COLS), jnp.int32),
            pltpu.VMEM((CHUNK, EMBED_DIM), jnp.float32),
            pltpu.SemaphoreType.DMA,
        ],
    )
    def k(idx_hbm, table_hbm, emb_hbm, mask_hbm, idx_v, mask_v, rows_v, sem):
        wid = lax.axis_index("s") * 2 + lax.axis_index("c")
        r0 = wid * ROWS_PER_W
        pltpu.sync_copy(idx_hbm.at[wid], idx_v)

        def mask_row(r, carry):
            for c in range(IDX_COLS // 16):
                v = idx_v[r, pl.ds(c * 16, 16)]
                mask_v[r, pl.ds(c * 16, 16)] = jnp.minimum(
                    jnp.abs(v), jnp.full((16,), 1, jnp.int32)
                )
            return carry

        lax.fori_loop(0, ROWS_PER_W, mask_row, 0)
        pltpu.sync_copy(mask_v, mask_hbm.at[wid])

        def body(j, carry):
            pltpu.async_copy(table_hbm.at[idx_v.at[j]], rows_v, sem).wait()
            pltpu.sync_copy(
                rows_v, emb_hbm.at[pl.ds(r0 * IDX_COLS + j * CHUNK, CHUNK)]
            )
            return carry

        lax.fori_loop(0, ROWS_PER_W, body, 0)

    return k(idx2d, table)


def kernel(sen_idx, table):
    idx2d = sen_idx.reshape(NUM_WORKERS, ROWS_PER_W, IDX_COLS).astype(jnp.int32)
    emb, mask_i32 = _sc_embed(idx2d, table)
    sen_emb = emb.reshape(BATCH, SEQ, EMBED_DIM)
    mask = (mask_i32 != 0).reshape(BATCH, SEQ)
    return (sen_emb, mask)


# speedup vs baseline: 3.3378x; 1.1283x over previous
import functools

import jax
import jax.numpy as jnp
from jax import lax
from jax.experimental import pallas as pl
from jax.experimental.pallas import tpu as pltpu
from jax.experimental.pallas import tpu_sc as plsc

EMBED_DIM = 128
SEQ = 50
BATCH = 4096
B_TOTAL = BATCH * SEQ
IDX_COLS = 128
NUM_WORKERS = 32
ROWS_PER_W = B_TOTAL // IDX_COLS // NUM_WORKERS
CHUNK = 128
NBUF = 5


def _sc_embed(idx3d, table):
    mesh = plsc.VectorSubcoreMesh(core_axis_name="c", subcore_axis_name="s")

    @functools.partial(
        pl.kernel,
        mesh=mesh,
        out_type=[
            jax.ShapeDtypeStruct((B_TOTAL, EMBED_DIM), jnp.float32),
            jax.ShapeDtypeStruct((NUM_WORKERS, ROWS_PER_W, IDX_COLS), jnp.int32),
        ],
        scratch_types=(
            [pltpu.VMEM((ROWS_PER_W, IDX_COLS), jnp.int32),
             pltpu.VMEM((ROWS_PER_W, IDX_COLS), jnp.int32)]
            + [pltpu.VMEM((CHUNK, EMBED_DIM), jnp.float32) for _ in range(NBUF)]
            + [pltpu.SemaphoreType.DMA for _ in range(2 * NBUF)]
        ),
    )
    def k(idx_hbm, table_hbm, emb_hbm, mask_hbm, idx_v, mask_v, *bufs_sems):
        rows = bufs_sems[:NBUF]
        gsem = bufs_sems[NBUF:2 * NBUF]
        osem = bufs_sems[2 * NBUF:]
        wid = lax.axis_index("s") * 2 + lax.axis_index("c")
        r0 = wid * ROWS_PER_W

        def out_slice(c):
            return emb_hbm.at[pl.ds((r0 + c) * CHUNK, CHUNK)]

        def gather_start(c, b):
            pltpu.async_copy(table_hbm.at[idx_v.at[c]], rows[b], gsem[b])

        def gather_wait(c, b):
            pltpu.make_async_copy(
                table_hbm.at[idx_v.at[c]], rows[b], gsem[b]
            ).wait()

        def out_start(c, b):
            pltpu.async_copy(rows[b], out_slice(c), osem[b])

        def out_wait(c, b):
            pltpu.make_async_copy(rows[b], out_slice(c), osem[b]).wait()

        pltpu.sync_copy(idx_hbm.at[wid], idx_v)
        for b in range(NBUF):
            gather_start(b, b)

        def mask_row(r, carry):
            for c in range(IDX_COLS // 16):
                v = idx_v[r, pl.ds(c * 16, 16)]
                mask_v[r, pl.ds(c * 16, 16)] = jnp.minimum(
                    jnp.abs(v), jnp.full((16,), 1, jnp.int32)
                )
            return carry

        lax.fori_loop(0, ROWS_PER_W, mask_row, 0)
        pltpu.sync_copy(mask_v, mask_hbm.at[wid])

        def outer(t, carry):
            for b in range(NBUF):
                j = t * NBUF + b
                gather_wait(j, b)
                out_start(j, b)
                pb = (b - 1) % NBUF
                pj = j + NBUF - 1

                @pl.when((j > 0) & (pj < ROWS_PER_W))
                def _():
                    out_wait(j - 1, pb)
                    gather_start(pj, pb)

            return carry

        lax.fori_loop(0, ROWS_PER_W // NBUF, outer, 0)
        out_wait(ROWS_PER_W - 1, NBUF - 1)

    return k(idx3d, table)


def kernel(sen_idx, table):
    idx3d = sen_idx.reshape(NUM_WORKERS, ROWS_PER_W, IDX_COLS).astype(jnp.int32)
    emb, mask_i32 = _sc_embed(idx3d, table)
    sen_emb = emb.reshape(BATCH, SEQ, EMBED_DIM)
    mask = (mask_i32 != 0).reshape(BATCH, SEQ)
    return (sen_emb, mask)


# speedup vs baseline: 5.8971x; 1.7667x over previous
import functools

import jax
import jax.numpy as jnp
from jax import lax
from jax.experimental import pallas as pl
from jax.experimental.pallas import tpu as pltpu
from jax.experimental.pallas import tpu_sc as plsc

EMBED_DIM = 128
SEQ = 50
BATCH = 4096
NUM_WORKERS = 32
SEQ_PER_W = BATCH // NUM_WORKERS
NBUF = 8


def _sc_embed(sen_idx, table):
    mesh = plsc.VectorSubcoreMesh(core_axis_name="c", subcore_axis_name="s")

    @functools.partial(
        pl.kernel,
        mesh=mesh,
        out_type=[
            jax.ShapeDtypeStruct((BATCH, SEQ, EMBED_DIM), jnp.float32),
            jax.ShapeDtypeStruct((BATCH, SEQ), jnp.int32),
        ],
        scratch_types=(
            [pltpu.VMEM((SEQ_PER_W, SEQ), jnp.int32),
             pltpu.VMEM((SEQ_PER_W, SEQ), jnp.int32)]
            + [pltpu.VMEM((SEQ, EMBED_DIM), jnp.float32) for _ in range(NBUF)]
            + [pltpu.SemaphoreType.DMA for _ in range(2 * NBUF)]
        ),
    )
    def k(idx_hbm, table_hbm, emb_hbm, mask_hbm, idx_v, mask_v, *bufs_sems):
        rows = bufs_sems[:NBUF]
        gsem = bufs_sems[NBUF:2 * NBUF]
        osem = bufs_sems[2 * NBUF:]
        wid = lax.axis_index("s") * 2 + lax.axis_index("c")
        s0 = wid * SEQ_PER_W

        def gather_start(s, b):
            pltpu.async_copy(table_hbm.at[idx_v.at[s]], rows[b], gsem[b])

        def gather_wait(s, b):
            pltpu.make_async_copy(
                table_hbm.at[idx_v.at[s]], rows[b], gsem[b]
            ).wait()

        def out_start(s, b):
            pltpu.async_copy(rows[b], emb_hbm.at[s0 + s], osem[b])

        def out_wait(s, b):
            pltpu.make_async_copy(rows[b], emb_hbm.at[s0 + s], osem[b]).wait()

        pltpu.sync_copy(idx_hbm.at[pl.ds(s0, SEQ_PER_W)], idx_v)
        for b in range(NBUF):
            gather_start(b, b)

        def mask_row(r, carry):
            for c in (0, 16, 32, 34):
                v = idx_v[r, pl.ds(c, 16)]
                mask_v[r, pl.ds(c, 16)] = jnp.minimum(
                    jnp.abs(v), jnp.full((16,), 1, jnp.int32)
                )
            return carry

        lax.fori_loop(0, SEQ_PER_W, mask_row, 0)
        pltpu.sync_copy(mask_v, mask_hbm.at[pl.ds(s0, SEQ_PER_W)])

        def outer(t, carry):
            for b in range(NBUF):
                s = t * NBUF + b
                gather_wait(s, b)
                out_start(s, b)
                pb = (b - 1) % NBUF
                ps = s + NBUF - 1

                @pl.when((s > 0) & (ps < SEQ_PER_W))
                def _():
                    out_wait(s - 1, pb)
                    gather_start(ps, pb)

            return carry

        lax.fori_loop(0, SEQ_PER_W // NBUF, outer, 0)
        out_wait(SEQ_PER_W - 1, NBUF - 1)

    return k(sen_idx, table)


def kernel(sen_idx, table):
    emb, mask_i32 = _sc_embed(sen_idx.astype(jnp.int32), table)
    return (emb, mask_i32 != 0)


# speedup vs baseline: 10.3547x; 1.7559x over previous
import functools

import jax
import jax.numpy as jnp
from jax import lax
from jax.experimental import pallas as pl
from jax.experimental.pallas import tpu as pltpu
from jax.experimental.pallas import tpu_sc as plsc

EMBED_DIM = 128
SEQ = 50
BATCH = 4096
NUM_WORKERS = 32
BLK = BATCH // NUM_WORKERS
NBUF = 5


def _sc_embed(idx_t, table):
    mesh = plsc.VectorSubcoreMesh(core_axis_name="c", subcore_axis_name="s")

    @functools.partial(
        pl.kernel,
        mesh=mesh,
        out_type=[
            jax.ShapeDtypeStruct((SEQ, BATCH, EMBED_DIM), jnp.float32),
            jax.ShapeDtypeStruct((SEQ, BATCH), jnp.int32),
        ],
        scratch_types=(
            [pltpu.VMEM((SEQ, BLK), jnp.int32),
             pltpu.VMEM((SEQ, BLK), jnp.int32)]
            + [pltpu.VMEM((BLK, EMBED_DIM), jnp.float32) for _ in range(NBUF)]
            + [pltpu.SemaphoreType.DMA for _ in range(2 * NBUF)]
        ),
    )
    def k(idx_hbm, table_hbm, emb_hbm, mask_hbm, idx_v, mask_v, *bufs_sems):
        rows = bufs_sems[:NBUF]
        gsem = bufs_sems[NBUF:2 * NBUF]
        osem = bufs_sems[2 * NBUF:]
        wid = lax.axis_index("s") * 2 + lax.axis_index("c")
        n0 = wid * BLK

        def gather_start(p, b):
            pltpu.async_copy(table_hbm.at[idx_v.at[p]], rows[b], gsem[b])

        def gather_wait(p, b):
            pltpu.make_async_copy(
                table_hbm.at[idx_v.at[p]], rows[b], gsem[b]
            ).wait()

        def out_start(p, b):
            pltpu.async_copy(rows[b], emb_hbm.at[p, pl.ds(n0, BLK)], osem[b])

        def out_wait(p, b):
            pltpu.make_async_copy(
                rows[b], emb_hbm.at[p, pl.ds(n0, BLK)], osem[b]
            ).wait()

        pltpu.sync_copy(idx_hbm.at[pl.ds(0, SEQ), pl.ds(n0, BLK)], idx_v)
        for b in range(NBUF):
            gather_start(b, b)

        def mask_row(r, carry):
            for c in range(BLK // 16):
                v = idx_v[r, pl.ds(c * 16, 16)]
                mask_v[r, pl.ds(c * 16, 16)] = jnp.minimum(
                    jnp.abs(v), jnp.full((16,), 1, jnp.int32)
                )
            return carry

        lax.fori_loop(0, SEQ, mask_row, 0)
        pltpu.sync_copy(mask_v, mask_hbm.at[pl.ds(0, SEQ), pl.ds(n0, BLK)])

        def outer(t, carry):
            for b in range(NBUF):
                p = t * NBUF + b
                gather_wait(p, b)
                out_start(p, b)
                pb = (b - 1) % NBUF
                pp = p + NBUF - 1

                @pl.when((p > 0) & (pp < SEQ))
                def _():
                    out_wait(p - 1, pb)
                    gather_start(pp, pb)

            return carry

        lax.fori_loop(0, SEQ // NBUF, outer, 0)
        out_wait(SEQ - 1, NBUF - 1)

    return k(idx_t, table)


def kernel(sen_idx, table):
    idx_t = sen_idx.astype(jnp.int32).T
    emb, mask_i32 = _sc_embed(idx_t, table)
    sen_emb = emb.transpose(1, 0, 2)
    mask = (mask_i32 != 0).T
    return (sen_emb, mask)
